# Initial kernel scaffold; baseline (speedup 1.0000x reference)
import functools

import jax
import jax.numpy as jnp
from jax import lax
from jax.experimental import pallas as pl
from jax.experimental.pallas import tpu as pltpu
from jax.experimental.pallas import tpu_sc as plsc

_N = 10000
_D = 128
_E = 320000
_NC = 2
_NS = 16
_NW = _NC * _NS
_EPW = _E // _NW
_CHUNK = 80
_NCHUNK = _EPW // _CHUNK
_RPT = _N // _NS

_BLK = 2000

_mesh = plsc.VectorSubcoreMesh(core_axis_name="c", subcore_axis_name="s")


@functools.partial(
    pl.kernel,
    out_type=jax.ShapeDtypeStruct((_NC, _N, _D), jnp.float32),
    mesh=_mesh,
    scratch_types=[
        pltpu.VMEM((_NCHUNK, _CHUNK), jnp.int32),
        pltpu.VMEM((_NCHUNK, _CHUNK), jnp.int32),
        pltpu.VMEM((_CHUNK, _D), jnp.float32),
        pltpu.VMEM_SHARED((_N, _D), jnp.float32),
        pltpu.SemaphoreType.DMA,
    ],
)
def _agg_sc(h_hbm, src_hbm, dst_hbm, zero_hbm, out_hbm,
            src_v, dst_v, rows_v, acc_sh, sem):
    c = lax.axis_index("c")
    s = lax.axis_index("s")
    wid = c * _NS + s
    pltpu.sync_copy(zero_hbm, acc_sh.at[pl.ds(s * _RPT, _RPT)])
    pltpu.sync_copy(src_hbm.at[wid], src_v)
    pltpu.sync_copy(dst_hbm.at[wid], dst_v)
    plsc.subcore_barrier()

    def body(j, carry):
        pltpu.async_copy(h_hbm.at[src_v.at[j]], rows_v, sem).wait()
        pltpu.sync_copy(rows_v, acc_sh.at[dst_v.at[j]], add=True)
        return carry

    lax.fori_loop(0, _NCHUNK, body, 0)
    plsc.subcore_barrier()
    pltpu.sync_copy(acc_sh.at[pl.ds(s * _RPT, _RPT)],
                    out_hbm.at[c, pl.ds(s * _RPT, _RPT)])


def _full(shape):
    return pl.BlockSpec(shape, lambda i: (0,) * len(shape))


def _rows(shape):
    return pl.BlockSpec(shape, lambda i: (i,) + (0,) * (len(shape) - 1))


def _dot(a, b):
    return jnp.dot(a, b, precision=lax.Precision.HIGHEST,
                   preferred_element_type=jnp.float32)


def _pre_body(x_ref, wa_ref, ba_ref, wb_ref, o_ref):
    h0 = _dot(x_ref[...], wa_ref[...]) + ba_ref[...]
    o_ref[...] = _dot(h0, wb_ref[...])


def _pre_tc(x, wpre_t, b_pre, w1_t):
    return pl.pallas_call(
        _pre_body,
        grid=(_N // _BLK,),
        in_specs=[_rows((_BLK, _D)), _full((_D, _D)), _full((1, _D)),
                  _full((_D, _D))],
        out_specs=_rows((_BLK, _D)),
        out_shape=jax.ShapeDtypeStruct((_N, _D), jnp.float32),
    )(x, wpre_t, b_pre, w1_t)


def _mid_body(p_ref, a0_ref, a1_ref, b_ref, w_ref, o_ref):
    h = jnp.maximum(p_ref[...] + a0_ref[...] + a1_ref[...] + b_ref[...], 0.0)
    o_ref[...] = _dot(h, w_ref[...])


def _mid_tc(p, a0, a1, b1, w2_t):
    return pl.pallas_call(
        _mid_body,
        grid=(_N // _BLK,),
        in_specs=[_rows((_BLK, _D)), _rows((_BLK, _D)), _rows((_BLK, _D)),
                  _full((1, _D)), _full((_D, _D))],
        out_specs=_rows((_BLK, _D)),
        out_shape=jax.ShapeDtypeStruct((_N, _D), jnp.float32),
    )(p, a0, a1, b1, w2_t)


def _out_body(q_ref, a0_ref, a1_ref, b_ref, o_ref):
    o_ref[...] = q_ref[...] + a0_ref[...] + a1_ref[...] + b_ref[...]


def _out_tc(q, a0, a1, b2):
    return pl.pallas_call(
        _out_body,
        grid=(_N // _BLK,),
        in_specs=[_rows((_BLK, _D)), _rows((_BLK, _D)), _rows((_BLK, _D)),
                  _full((1, _D))],
        out_specs=_rows((_BLK, _D)),
        out_shape=jax.ShapeDtypeStruct((_N, _D), jnp.float32),
    )(q, a0, a1, b2)


def kernel(x, edge_index, W_pre, b_pre, W1, b1, W2, b2):
    src = edge_index[0].reshape(_NW, _NCHUNK, _CHUNK)
    dst = edge_index[1].reshape(_NW, _NCHUNK, _CHUNK)
    zeros = jnp.zeros((_RPT, _D), jnp.float32)

    p = _pre_tc(x, W_pre.T, b_pre.reshape(1, _D), W1.T)
    parts = _agg_sc(p, src, dst, zeros)
    q = _mid_tc(p, parts[0], parts[1], b1.reshape(1, _D), W2.T)
    parts2 = _agg_sc(q, src, dst, zeros)
    return _out_tc(q, parts2[0], parts2[1], b2.reshape(1, _D))


# speedup vs baseline: 6.2472x; 6.2472x over previous
import functools

import jax
import jax.numpy as jnp
from jax import lax
from jax.experimental import pallas as pl
from jax.experimental.pallas import tpu as pltpu
from jax.experimental.pallas import tpu_sc as plsc

_N = 10000
_D = 128
_E = 320000
_NC = 2
_NS = 16
_NW = _NC * _NS
_EPW = _E // _NW
_CHUNK = 80
_NCHUNK = _EPW // _CHUNK
_NP = 10240
_RPT = _NP // _NS

_BLK = 2000

def _agg_body(h_hbm, src_hbm, dst_hbm, zero_hbm, out_hbm,
              src_v, dst_v, rows_v, acc_sh, sem):
    c = lax.axis_index("c")
    s = lax.axis_index("s")
    wid = c * _NS + s
    pltpu.sync_copy(zero_hbm, acc_sh.at[pl.ds(s * _RPT, _RPT)])
    pltpu.sync_copy(src_hbm.at[wid], src_v)
    pltpu.sync_copy(dst_hbm.at[wid], dst_v)
    plsc.subcore_barrier()

    def body(j, carry):
        pltpu.async_copy(h_hbm.at[src_v.at[j]], rows_v, sem).wait()
        pltpu.sync_copy(rows_v, acc_sh.at[dst_v.at[j]], add=True)
        return carry

    lax.fori_loop(0, _NCHUNK, body, 0)
    plsc.subcore_barrier()
    pltpu.sync_copy(acc_sh.at[pl.ds(s * _RPT, _RPT)],
                    out_hbm.at[c, pl.ds(s * _RPT, _RPT)])


@functools.lru_cache(maxsize=None)
def _agg_sc_kernel():
    mesh = plsc.VectorSubcoreMesh(core_axis_name="c", subcore_axis_name="s",
                                  num_cores=_NC, num_subcores=_NS)
    return pl.kernel(
        _agg_body,
        out_type=jax.ShapeDtypeStruct((_NC, _NP, _D), jnp.float32),
        mesh=mesh,
        scratch_types=[
            pltpu.VMEM((_NCHUNK, _CHUNK), jnp.int32),
            pltpu.VMEM((_NCHUNK, _CHUNK), jnp.int32),
            pltpu.VMEM((_CHUNK, _D), jnp.float32),
            pltpu.VMEM_SHARED((_NP, _D), jnp.float32),
            pltpu.SemaphoreType.DMA,
        ],
    )


def _full(shape):
    return pl.BlockSpec(shape, lambda i: (0,) * len(shape))


def _rows(shape):
    return pl.BlockSpec(shape, lambda i: (i,) + (0,) * (len(shape) - 1))


def _dot(a, b):
    return jnp.dot(a, b, precision=lax.Precision.HIGHEST,
                   preferred_element_type=jnp.float32)


def _pre_body(x_ref, wa_ref, ba_ref, wb_ref, o_ref):
    h0 = _dot(x_ref[...], wa_ref[...]) + ba_ref[...]
    o_ref[...] = _dot(h0, wb_ref[...])


def _pre_tc(x, wpre_t, b_pre, w1_t):
    return pl.pallas_call(
        _pre_body,
        grid=(_N // _BLK,),
        in_specs=[_rows((_BLK, _D)), _full((_D, _D)), _full((1, _D)),
                  _full((_D, _D))],
        out_specs=_rows((_BLK, _D)),
        out_shape=jax.ShapeDtypeStruct((_N, _D), jnp.float32),
    )(x, wpre_t, b_pre, w1_t)


def _mid_body(p_ref, a0_ref, a1_ref, b_ref, w_ref, o_ref):
    h = jnp.maximum(p_ref[...] + a0_ref[...] + a1_ref[...] + b_ref[...], 0.0)
    o_ref[...] = _dot(h, w_ref[...])


def _mid_tc(p, a0, a1, b1, w2_t):
    return pl.pallas_call(
        _mid_body,
        grid=(_N // _BLK,),
        in_specs=[_rows((_BLK, _D)), _rows((_BLK, _D)), _rows((_BLK, _D)),
                  _full((1, _D)), _full((_D, _D))],
        out_specs=_rows((_BLK, _D)),
        out_shape=jax.ShapeDtypeStruct((_N, _D), jnp.float32),
    )(p, a0, a1, b1, w2_t)


def _out_body(q_ref, a0_ref, a1_ref, b_ref, o_ref):
    o_ref[...] = q_ref[...] + a0_ref[...] + a1_ref[...] + b_ref[...]


def _out_tc(q, a0, a1, b2):
    return pl.pallas_call(
        _out_body,
        grid=(_N // _BLK,),
        in_specs=[_rows((_BLK, _D)), _rows((_BLK, _D)), _rows((_BLK, _D)),
                  _full((1, _D))],
        out_specs=_rows((_BLK, _D)),
        out_shape=jax.ShapeDtypeStruct((_N, _D), jnp.float32),
    )(q, a0, a1, b2)


def kernel(x, edge_index, W_pre, b_pre, W1, b1, W2, b2):
    src = edge_index[0].reshape(_NW, _NCHUNK, _CHUNK)
    dst = edge_index[1].reshape(_NW, _NCHUNK, _CHUNK)
    zeros = jnp.zeros((_RPT, _D), jnp.float32)

    p = _pre_tc(x, W_pre.T, b_pre.reshape(1, _D), W1.T)
    agg = _agg_sc_kernel()
    parts = agg(p, src, dst, zeros)
    q = _mid_tc(p, parts[0], parts[1], b1.reshape(1, _D), W2.T)
    parts2 = agg(q, src, dst, zeros)
    return _out_tc(q, parts2[0], parts2[1], b2.reshape(1, _D))
